# baseline (speedup 1.0000x reference)
import functools

import jax
import jax.numpy as jnp
from jax import lax
from jax.experimental import pallas as pl
from jax.experimental.pallas import tpu as pltpu
from jax.experimental.pallas import tpu_sc as plsc

WEIGHT_DECAY = 0.001
NCORES = 2
NSUB = 16
NW = NCORES * NSUB
CH = 128
FIRE = 8
PB = 512


def _sc_gather(in_embed, out_embed, idx2):
    m2, ch = idx2.shape
    m = m2 * ch
    d = in_embed.shape[1]
    per_w = m2 // NW
    groups = per_w // FIRE
    buf_rows = FIRE * ch

    mesh = plsc.VectorSubcoreMesh(core_axis_name="c", subcore_axis_name="s",
                                  num_cores=NCORES, num_subcores=NSUB)

    @functools.partial(
        pl.kernel,
        mesh=mesh,
        compiler_params=pltpu.CompilerParams(use_tc_tiling_on_sc=False),
        out_type=(jax.ShapeDtypeStruct((m, d), jnp.float32),
                  jax.ShapeDtypeStruct((m, d), jnp.float32)),
        scratch_types=[
            pltpu.VMEM((per_w, ch), jnp.int32),
            pltpu.VMEM((buf_rows, d), jnp.float32),
            pltpu.SemaphoreType.DMA,
        ],
    )
    def gather_kernel(in_hbm, out_hbm, idx_hbm, g_in, g_out, idx_v, rows_v, sem):
        wid = lax.axis_index("s") * NCORES + lax.axis_index("c")
        pltpu.sync_copy(idx_hbm.at[pl.ds(wid * per_w, per_w)], idx_v)
        for tbl, g in ((in_hbm, g_in), (out_hbm, g_out)):
            for grp in range(groups):
                for j in range(FIRE):
                    pltpu.make_async_copy(
                        tbl.at[idx_v.at[grp * FIRE + j]],
                        rows_v.at[pl.ds(j * ch, ch)], sem).start()
                for j in range(FIRE):
                    pltpu.make_async_copy(
                        tbl.at[idx_v.at[grp * FIRE + j]],
                        rows_v.at[pl.ds(j * ch, ch)], sem).wait()
                base = (wid * per_w + grp * FIRE) * ch
                pltpu.sync_copy(rows_v, g.at[pl.ds(base, buf_rows)])

    return gather_kernel(in_embed, out_embed, idx2)


def _tc_compute(g_in, g_out, edge_w2, b, w_win, ns):
    d = g_in.shape[1]
    bw = b * w_win
    nj = b // PB
    m = g_in.shape[0]
    ids_blk = (m - b) // PB
    npb = ns * PB
    un_blk = bw // npb
    vn_blk = (bw + bw * ns) // npb

    def body(ui_r, vi_r, nui_r, nvi_r, uo_r, vo_r, nuo_r, nvo_r, w_r, out_r):
        w = w_r[0, :]
        ui = ui_r[...]
        vi = vi_r[...]
        uo = uo_r[...]
        vo = vo_r[...]
        nui = nui_r[...]
        nvi = nvi_r[...]
        nuo = nuo_r[...]
        nvo = nvo_r[...]

        def logsig(x):
            return jnp.log(jax.nn.sigmoid(jnp.clip(x, -6.0, 6.0)))

        lt = jnp.sum(logsig(jnp.sum(ui * vi * w, axis=-1)))
        lt += jnp.sum(logsig(jnp.sum(uo * vo * w, axis=-1)))

        w3 = w.reshape(1, 1, d)

        def sterm(n, x):
            n3 = n.reshape(PB, ns, d)
            x3 = x.reshape(PB, 1, d)
            return jnp.sum(logsig(-jnp.sum(n3 * x3 * w3, axis=-1)))

        s = sterm(nui, vi) + sterm(nuo, vo) + sterm(nvi, ui) + sterm(nvo, uo)

        sq = (jnp.sum(ui * ui) + jnp.sum(vi * vi)
              + jnp.sum(uo * uo) + jnp.sum(vo * vo)
              + jnp.sum(nui * nui) + jnp.sum(nuo * nuo)
              + jnp.sum(nvi * nvi) + jnp.sum(nvo * nvo))

        contrib = -(lt + 0.5 * s - 0.5 * WEIGHT_DECAY * sq) / b

        @pl.when((pl.program_id(0) == 0) & (pl.program_id(1) == 0))
        def _():
            out_r[0, 0] = 0.0

        out_r[0, 0] += contrib

    row_specs = [
        pl.BlockSpec((PB, d), lambda wi, j: (ids_blk + j, 0)),
        pl.BlockSpec((PB, d), lambda wi, j: (wi * nj + j, 0)),
        pl.BlockSpec((npb, d), lambda wi, j: (un_blk + wi * nj + j, 0)),
        pl.BlockSpec((npb, d), lambda wi, j: (vn_blk + wi * nj + j, 0)),
    ]
    return pl.pallas_call(
        body,
        grid=(w_win, nj),
        in_specs=row_specs + row_specs + [
            pl.BlockSpec((1, d), lambda wi, j: (0, 0))],
        out_specs=pl.BlockSpec(memory_space=pltpu.SMEM),
        out_shape=jax.ShapeDtypeStruct((1, 1), jnp.float32),
    )(g_in, g_in, g_in, g_in, g_out, g_out, g_out, g_out, edge_w2)


def kernel(input_labels, out_labels, in_embed, out_embed, edge_w,
           u_noise, v_noise, num_sampled):
    del num_sampled
    b, w1 = out_labels.shape
    w_win = w1 - 1
    d = in_embed.shape[1]
    ns = u_noise.shape[1]

    ids = input_labels[:, 1].astype(jnp.int32)
    out_t = out_labels[:, 1:].reshape(-1).astype(jnp.int32)
    idx = jnp.concatenate([out_t,
                           u_noise.reshape(-1).astype(jnp.int32),
                           v_noise.reshape(-1).astype(jnp.int32),
                           ids])
    idx2 = idx.reshape(idx.shape[0] // CH, CH)

    g_in, g_out = _sc_gather(in_embed, out_embed, idx2)
    res = _tc_compute(g_in, g_out, edge_w.reshape(1, d), b, w_win, ns)
    return res[0, 0]


# speedup vs baseline: 1.3035x; 1.3035x over previous
import functools

import jax
import jax.numpy as jnp
from jax import lax
from jax.experimental import pallas as pl
from jax.experimental.pallas import tpu as pltpu
from jax.experimental.pallas import tpu_sc as plsc

WEIGHT_DECAY = 0.001
NCORES = 2
NSUB = 16
NW = NCORES * NSUB
CH = 128
FIRE = 8
PB = 1024


def _sc_gather(in_embed, out_embed, idx2):
    m2, ch = idx2.shape
    m = m2 * ch
    d = in_embed.shape[1]
    per_w = m2 // NW
    groups = per_w // FIRE
    buf_rows = FIRE * ch

    mesh = plsc.VectorSubcoreMesh(core_axis_name="c", subcore_axis_name="s",
                                  num_cores=NCORES, num_subcores=NSUB)

    @functools.partial(
        pl.kernel,
        mesh=mesh,
        compiler_params=pltpu.CompilerParams(use_tc_tiling_on_sc=False),
        out_type=(jax.ShapeDtypeStruct((m, d), jnp.float32),
                  jax.ShapeDtypeStruct((m, d), jnp.float32)),
        scratch_types=[
            pltpu.VMEM((per_w, ch), jnp.int32),
            pltpu.VMEM((buf_rows, d), jnp.float32),
            pltpu.SemaphoreType.DMA,
        ],
    )
    def gather_kernel(in_hbm, out_hbm, idx_hbm, g_in, g_out, idx_v, rows_v, sem):
        wid = lax.axis_index("s") * NCORES + lax.axis_index("c")
        pltpu.sync_copy(idx_hbm.at[pl.ds(wid * per_w, per_w)], idx_v)
        for tbl, g in ((in_hbm, g_in), (out_hbm, g_out)):
            for grp in range(groups):
                for j in range(FIRE):
                    pltpu.make_async_copy(
                        tbl.at[idx_v.at[grp * FIRE + j]],
                        rows_v.at[pl.ds(j * ch, ch)], sem).start()
                for j in range(FIRE):
                    pltpu.make_async_copy(
                        tbl.at[idx_v.at[grp * FIRE + j]],
                        rows_v.at[pl.ds(j * ch, ch)], sem).wait()
                base = (wid * per_w + grp * FIRE) * ch
                pltpu.sync_copy(rows_v, g.at[pl.ds(base, buf_rows)])

    return gather_kernel(in_embed, out_embed, idx2)


def _tc_compute(g_in, g_out, edge_w2, b, w_win, ns):
    d = g_in.shape[1]
    bw = b * w_win
    nj = b // PB
    m = g_in.shape[0]
    ids_blk = (m - b) // PB
    un_blk = bw // PB
    vn_blk = (bw + bw * ns) // PB
    bw_blk = bw // PB

    def body(*refs):
        ui_r, vi_r, uo_r, vo_r = refs[0:4]
        nui_rs = refs[4:4 + ns]
        nvi_rs = refs[4 + ns:4 + 2 * ns]
        nuo_rs = refs[4 + 2 * ns:4 + 3 * ns]
        nvo_rs = refs[4 + 3 * ns:4 + 4 * ns]
        w_r = refs[4 + 4 * ns]
        out_r = refs[5 + 4 * ns]

        w = w_r[0, :]
        ui = ui_r[...]
        vi = vi_r[...]
        uo = uo_r[...]
        vo = vo_r[...]
        viw = vi * w
        vow = vo * w
        uiw = ui * w
        uow = uo * w

        def logsig(x):
            return jnp.log(jax.nn.sigmoid(jnp.clip(x, -6.0, 6.0)))

        acc = jnp.sum(logsig(jnp.sum(ui * viw, axis=-1)))
        acc += jnp.sum(logsig(jnp.sum(uo * vow, axis=-1)))

        sq = (jnp.sum(ui * ui) + jnp.sum(vi * vi)
              + jnp.sum(uo * uo) + jnp.sum(vo * vo))
        s = jnp.float32(0.0)
        for nrs, base in ((nui_rs, viw), (nuo_rs, vow),
                          (nvi_rs, uiw), (nvo_rs, uow)):
            for k in range(ns):
                nk = nrs[k][...]
                s += jnp.sum(logsig(-jnp.sum(nk * base, axis=-1)))
                sq += jnp.sum(nk * nk)

        contrib = -(acc + 0.5 * s - 0.5 * WEIGHT_DECAY * sq) / b

        @pl.when((pl.program_id(0) == 0) & (pl.program_id(1) == 0))
        def _():
            out_r[0, 0] = 0.0

        out_r[0, 0] += contrib

    def blk(off_blocks):
        return pl.BlockSpec(
            (PB, d), lambda wi, j, o=off_blocks: (o + wi * nj + j, 0))

    base_specs = [
        pl.BlockSpec((PB, d), lambda wi, j: (ids_blk + j, 0)),
        blk(0),
    ]
    nui_specs = [blk(un_blk + k * bw_blk) for k in range(ns)]
    nvi_specs = [blk(vn_blk + k * bw_blk) for k in range(ns)]

    in_specs = ([base_specs[0], base_specs[1], base_specs[0], base_specs[1]]
                + nui_specs + nvi_specs + nui_specs + nvi_specs
                + [pl.BlockSpec((1, d), lambda wi, j: (0, 0))])
    operands = ([g_in, g_in, g_out, g_out]
                + [g_in] * ns + [g_in] * ns + [g_out] * ns + [g_out] * ns
                + [edge_w2])
    return pl.pallas_call(
        body,
        grid=(w_win, nj),
        in_specs=in_specs,
        out_specs=pl.BlockSpec(memory_space=pltpu.SMEM),
        out_shape=jax.ShapeDtypeStruct((1, 1), jnp.float32),
    )(*operands)


def kernel(input_labels, out_labels, in_embed, out_embed, edge_w,
           u_noise, v_noise, num_sampled):
    del num_sampled
    b, w1 = out_labels.shape
    w_win = w1 - 1
    d = in_embed.shape[1]
    ns = u_noise.shape[1]

    ids = input_labels[:, 1].astype(jnp.int32)
    out_t = out_labels[:, 1:].reshape(-1).astype(jnp.int32)
    idx = jnp.concatenate([out_t,
                           u_noise.T.reshape(-1).astype(jnp.int32),
                           v_noise.T.reshape(-1).astype(jnp.int32),
                           ids])
    idx2 = idx.reshape(idx.shape[0] // CH, CH)

    g_in, g_out = _sc_gather(in_embed, out_embed, idx2)
    res = _tc_compute(g_in, g_out, edge_w.reshape(1, d), b, w_win, ns)
    return res[0, 0]
